# Initial kernel scaffold; baseline (speedup 1.0000x reference)
import jax
import jax.numpy as jnp
from jax.experimental import pallas as pl


def kernel(values, scenepoint_features, view_features, global_features, cam_idx, pt_idx, W, b):
    raise NotImplementedError("write your pallas kernel here")


# speedup vs baseline: 3.0600x; 3.0600x over previous
import functools

import jax
import jax.numpy as jnp
from jax import lax
from jax.experimental import pallas as pl
from jax.experimental.pallas import tpu as pltpu
from jax.experimental.pallas import tpu_sc as plsc

E = 320000
N_PTS = 10000
N_VIEWS = 500
D = 128

NC = 2
NS = 16
NW = NC * NS
BPW = E // NW
C = 400

BE = 2560


def _sc_gather_pair(pt_tbl, vw_tbl, pt_idx, cam_idx):
    mesh = plsc.VectorSubcoreMesh(core_axis_name="c", subcore_axis_name="s")

    @functools.partial(
        pl.kernel,
        mesh=mesh,
        out_type=(
            jax.ShapeDtypeStruct((E, D), jnp.float32),
            jax.ShapeDtypeStruct((E, D), jnp.float32),
        ),
        scratch_types=[
            pltpu.VMEM((C,), jnp.int32),
            pltpu.VMEM((C,), jnp.int32),
            pltpu.VMEM((C, D), jnp.float32),
            pltpu.VMEM((C, D), jnp.float32),
            pltpu.SemaphoreType.DMA,
            pltpu.SemaphoreType.DMA,
        ],
    )
    def k(pt_hbm, vw_hbm, pi_hbm, ci_hbm, po_hbm, vo_hbm,
          pi_v, ci_v, rp_v, rv_v, sem1, sem2):
        wid = lax.axis_index("s") * NC + lax.axis_index("c")
        base = wid * BPW

        @pl.loop(0, BPW, step=C)
        def _(off):
            s = base + off
            pltpu.sync_copy(pi_hbm.at[pl.ds(s, C)], pi_v)
            pltpu.sync_copy(ci_hbm.at[pl.ds(s, C)], ci_v)
            cp1 = pltpu.async_copy(pt_hbm.at[pi_v], rp_v, sem1)
            cp2 = pltpu.async_copy(vw_hbm.at[ci_v], rv_v, sem2)
            cp1.wait()
            cp2.wait()
            pltpu.sync_copy(rp_v, po_hbm.at[pl.ds(s, C)])
            pltpu.sync_copy(rv_v, vo_hbm.at[pl.ds(s, C)])

    return k(pt_tbl, vw_tbl, pt_idx, cam_idx)


def _tc_body(v_ref, p_ref, vw_ref, w_ref, bg_ref, o_ref):
    acc = lax.dot_general(
        v_ref[...], w_ref[...],
        (((1,), (1,)), ((), ())),
        preferred_element_type=jnp.float32,
    )
    o_ref[...] = (acc + p_ref[...] + vw_ref[...] + bg_ref[...]) * 0.25


def kernel(values, scenepoint_features, view_features, global_features,
           cam_idx, pt_idx, W, b):
    pt_rows, vw_rows = _sc_gather_pair(
        scenepoint_features, view_features,
        pt_idx.astype(jnp.int32), cam_idx.astype(jnp.int32))

    bg = (b + global_features)[None, :]

    out = pl.pallas_call(
        _tc_body,
        grid=(E // BE,),
        in_specs=[
            pl.BlockSpec((BE, D), lambda i: (i, 0)),
            pl.BlockSpec((BE, D), lambda i: (i, 0)),
            pl.BlockSpec((BE, D), lambda i: (i, 0)),
            pl.BlockSpec((D, D), lambda i: (0, 0)),
            pl.BlockSpec((1, D), lambda i: (0, 0)),
        ],
        out_specs=pl.BlockSpec((BE, D), lambda i: (i, 0)),
        out_shape=jax.ShapeDtypeStruct((E, D), jnp.float32),
    )(values, pt_rows, vw_rows, W, bg)
    return out
